# baseline (speedup 1.0000x reference)
import functools

import jax
import jax.numpy as jnp
import numpy as np
from jax.experimental import pallas as pl
from jax.experimental.pallas import tpu as pltpu

_B = 64
_N = 4096
_D = 1024
_DE = 128
_TEMP = 1.0
_MIN_TEMPERATURE = 1e-05
_INV_TEMP = 1.0 / max(float(_TEMP), _MIN_TEMPERATURE)


def _tc_body(ns_ref, wp_ref, wsp_ref, bs_ref, ef_ref, ge_ref, gs_ref,
             lpf_ref, act_ref, sp_scr, stop_scr):
    b = pl.program_id(0)

    @pl.when(b == 0)
    def _init():
        sp_scr[...] = jnp.dot(ns_ref[...], wp_ref[...],
                              preferred_element_type=jnp.float32)
        stop_scr[...] = jnp.dot(ns_ref[...], wsp_ref[...],
                                preferred_element_type=jnp.float32) + bs_ref[0, 0]

    sp_b = sp_scr[pl.ds(b, 1), :]
    ef = ef_ref[0]
    x = jax.lax.dot_general(sp_b, ef, (((1,), (1,)), ((), ())),
                            preferred_element_type=jnp.float32)
    x = x * _INV_TEMP
    stop_l = stop_scr[pl.ds(b, 1), pl.ds(0, 1)] * _INV_TEMP

    m_e = jnp.max(x)
    m = jnp.maximum(m_e, stop_l[0, 0])
    lse = m + jnp.log(jnp.sum(jnp.exp(x - m)) + jnp.exp(stop_l[0, 0] - m))

    g_row = ge_ref[0]
    pert = x + g_row
    pm = jnp.max(pert)
    cols = jax.lax.broadcasted_iota(jnp.int32, (1, _N), 1)
    eidx = jnp.min(jnp.where(pert == pm, cols, _N))
    e_logit = jnp.max(jnp.where(cols == eidx, x, -jnp.inf))
    pert_stop = stop_l[0, 0] + gs_ref[0, 0, 0]
    take_stop = pert_stop > pm
    action = jnp.where(take_stop, _N, eidx)
    sel_logit = jnp.where(take_stop, stop_l[0, 0], e_logit)
    log_pf = sel_logit - lse

    lpf_ref[...] = jnp.full((1, 1, 128), log_pf, jnp.float32)
    act_ref[...] = jnp.full((1, 1, 128), action, jnp.int32)


@functools.partial(jax.jit, static_argnames=())
def _run(node_states, edge_feats, W_proj, W_stop_pad, b_stop_2d, g_edges, g_stop):
    grid = (_B,)
    out = pl.pallas_call(
        _tc_body,
        grid=grid,
        in_specs=[
            pl.BlockSpec((_B, _D), lambda b: (0, 0)),
            pl.BlockSpec((_D, _DE), lambda b: (0, 0)),
            pl.BlockSpec((_D, 128), lambda b: (0, 0)),
            pl.BlockSpec(memory_space=pltpu.SMEM),
            pl.BlockSpec((1, _N, _DE), lambda b: (b, 0, 0)),
            pl.BlockSpec((1, 1, _N), lambda b: (b, 0, 0)),
            pl.BlockSpec((1, 1, 128), lambda b: (b, 0, 0)),
        ],
        out_specs=[
            pl.BlockSpec((1, 1, 128), lambda b: (b, 0, 0)),
            pl.BlockSpec((1, 1, 128), lambda b: (b, 0, 0)),
        ],
        out_shape=[
            jax.ShapeDtypeStruct((_B, 1, 128), jnp.float32),
            jax.ShapeDtypeStruct((_B, 1, 128), jnp.int32),
        ],
        scratch_shapes=[
            pltpu.VMEM((_B, _DE), jnp.float32),
            pltpu.VMEM((_B, 128), jnp.float32),
        ],
    )(node_states, W_proj, W_stop_pad, b_stop_2d, edge_feats, g_edges, g_stop)
    return out[0][:, 0, 0], out[1][:, 0, 0]


def kernel(node_states, edge_feats, W_proj, W_stop, b_stop):
    u = jax.random.uniform(jax.random.key(1), (_B, _N + 1),
                           dtype=jnp.float32, minval=1e-9, maxval=1.0)
    gumbel = -jnp.log(-jnp.log(u))
    g_edges = gumbel[:, :_N].reshape(_B, 1, _N)
    g_stop = jnp.broadcast_to(gumbel[:, _N:].reshape(_B, 1, 1), (_B, 1, 128))
    W_stop_pad = jnp.pad(W_stop, ((0, 0), (0, 127)))
    b_stop_2d = b_stop.reshape(1, 1)
    return _run(node_states, edge_feats, W_proj, W_stop_pad, b_stop_2d,
                g_edges, g_stop)


# speedup vs baseline: 1.7857x; 1.7857x over previous
import functools

import jax
import jax.numpy as jnp
import numpy as np
from jax.experimental import pallas as pl
from jax.experimental.pallas import tpu as pltpu

_B = 64
_N = 4096
_D = 1024
_DE = 128
_G = 8
_TEMP = 1.0
_MIN_TEMPERATURE = 1e-05
_INV_TEMP = 1.0 / max(float(_TEMP), _MIN_TEMPERATURE)


def _tc_body(ns_ref, wp_ref, wsp_ref, bs_ref, ef_ref, ge_ref, gs_ref,
             lpf_ref, act_ref, sp_scr, stop_scr, x_scr):
    step = pl.program_id(0)

    @pl.when(step == 0)
    def _init():
        sp_scr[...] = jnp.dot(ns_ref[...], wp_ref[...],
                              preferred_element_type=jnp.float32)
        stop_scr[...] = jnp.dot(ns_ref[...], wsp_ref[...],
                                preferred_element_type=jnp.float32) + bs_ref[0, 0]

    base = step * _G
    for g in range(_G):
        sp_g = sp_scr[pl.ds(base + g, 1), :]
        x_scr[pl.ds(g, 1), :] = jax.lax.dot_general(
            sp_g, ef_ref[g], (((1,), (1,)), ((), ())),
            preferred_element_type=jnp.float32)

    x = x_scr[...] * _INV_TEMP
    stop_l = stop_scr[pl.ds(base, _G), pl.ds(0, 1)] * _INV_TEMP

    m = jnp.maximum(jnp.max(x, axis=1, keepdims=True), stop_l)
    lse = m + jnp.log(jnp.sum(jnp.exp(x - m), axis=1, keepdims=True)
                      + jnp.exp(stop_l - m))

    pert = x + ge_ref[...]
    pm = jnp.max(pert, axis=1, keepdims=True)
    cols = jax.lax.broadcasted_iota(jnp.int32, (_G, _N), 1)
    eidx = jnp.min(jnp.where(pert == pm, cols, _N), axis=1, keepdims=True)
    e_logit = jnp.max(jnp.where(cols == eidx, x, -jnp.inf),
                      axis=1, keepdims=True)
    pert_stop = stop_l + gs_ref[:, pl.ds(0, 1)]
    take_stop = pert_stop > pm
    action = jnp.where(take_stop, _N, eidx)
    log_pf = jnp.where(take_stop, stop_l, e_logit) - lse

    lpf_ref[...] = jnp.broadcast_to(log_pf, (_G, 128))
    act_ref[...] = jnp.broadcast_to(action, (_G, 128))


@jax.jit
def _run(node_states, edge_feats, W_proj, W_stop_pad, b_stop_2d, g_edges, g_stop):
    grid = (_B // _G,)
    out = pl.pallas_call(
        _tc_body,
        grid=grid,
        in_specs=[
            pl.BlockSpec((_B, _D), lambda s: (0, 0)),
            pl.BlockSpec((_D, _DE), lambda s: (0, 0)),
            pl.BlockSpec((_D, 128), lambda s: (0, 0)),
            pl.BlockSpec(memory_space=pltpu.SMEM),
            pl.BlockSpec((_G, _N, _DE), lambda s: (s, 0, 0)),
            pl.BlockSpec((_G, _N), lambda s: (s, 0)),
            pl.BlockSpec((_G, 128), lambda s: (s, 0)),
        ],
        out_specs=[
            pl.BlockSpec((_G, 128), lambda s: (s, 0)),
            pl.BlockSpec((_G, 128), lambda s: (s, 0)),
        ],
        out_shape=[
            jax.ShapeDtypeStruct((_B, 128), jnp.float32),
            jax.ShapeDtypeStruct((_B, 128), jnp.int32),
        ],
        scratch_shapes=[
            pltpu.VMEM((_B, _DE), jnp.float32),
            pltpu.VMEM((_B, 128), jnp.float32),
            pltpu.VMEM((_G, _N), jnp.float32),
        ],
    )(node_states, W_proj, W_stop_pad, b_stop_2d, edge_feats, g_edges, g_stop)
    return out[0][:, 0], out[1][:, 0]


def kernel(node_states, edge_feats, W_proj, W_stop, b_stop):
    u = jax.random.uniform(jax.random.key(1), (_B, _N + 1),
                           dtype=jnp.float32, minval=1e-9, maxval=1.0)
    gumbel = -jnp.log(-jnp.log(u))
    g_edges = gumbel[:, :_N]
    g_stop = jnp.broadcast_to(gumbel[:, _N:], (_B, 128))
    W_stop_pad = jnp.pad(W_stop, ((0, 0), (0, 127)))
    b_stop_2d = b_stop.reshape(1, 1)
    return _run(node_states, edge_feats, W_proj, W_stop_pad, b_stop_2d,
                g_edges, g_stop)
